# Initial kernel scaffold; baseline (speedup 1.0000x reference)
import functools

import jax
import jax.numpy as jnp
from jax import lax
from jax.experimental import pallas as pl
from jax.experimental.pallas import tpu as pltpu
from jax.experimental.pallas import tpu_sc as plsc

_VOCAB = 100000
_EMB = 64
_FIELD_D = 256
_HID = 128
_COMB = 64
_B = 4096
_L = 50
_TEMP = 0.5

_ROWBLK = 1024
_NBLK = 98
_NPAD = _NBLK * _ROWBLK
_PAD_ROW = _VOCAB + 1

_NC, _NS, _LANES = 2, 16, 16
_NW = _NC * _NS
_BPW = _B // _NW
_LP = 56
_CB = 2
_CHUNKS = _BPW // _CB
_CIDS = _CB * _LP


def _transform_body(tf_in, wf, bf, tl_in, wl, bl, tf_out, tl_out):
    i = pl.program_id(0)
    row = i * _ROWBLK + lax.broadcasted_iota(jnp.int32, (_ROWBLK, 1), 0)
    valid = row <= _VOCAB
    nonzero = row != 0
    xf = jnp.where(valid, tf_in[...], 0.0)
    hf = jnp.dot(xf, wf[...], preferred_element_type=jnp.float32) + bf[...]
    tf_out[...] = jnp.where(nonzero, jnp.maximum(hf, 0.0), 0.0)
    xl = jnp.where(valid, tl_in[...], 0.0)
    hl = jnp.dot(xl, wl[...], preferred_element_type=jnp.float32) + bl[...]
    tl_out[...] = jnp.where(nonzero, jnp.maximum(hl, 0.0), 0.0)


def _transform(field_table, field_Wp, field_bp2, lang_table, lang_Wp, lang_bp2):
    return pl.pallas_call(
        _transform_body,
        grid=(_NBLK,),
        in_specs=[
            pl.BlockSpec((_ROWBLK, _FIELD_D), lambda i: (i, 0)),
            pl.BlockSpec((_FIELD_D, _HID), lambda i: (0, 0)),
            pl.BlockSpec((1, _HID), lambda i: (0, 0)),
            pl.BlockSpec((_ROWBLK, _EMB), lambda i: (i, 0)),
            pl.BlockSpec((_EMB, _HID), lambda i: (0, 0)),
            pl.BlockSpec((1, _HID), lambda i: (0, 0)),
        ],
        out_specs=[
            pl.BlockSpec((_ROWBLK, _HID), lambda i: (i, 0)),
            pl.BlockSpec((_ROWBLK, _HID), lambda i: (i, 0)),
        ],
        out_shape=[
            jax.ShapeDtypeStruct((_NPAD, _HID), jnp.float32),
            jax.ShapeDtypeStruct((_NPAD, _HID), jnp.float32),
        ],
    )(field_table, field_Wp, field_bp2, lang_table, lang_Wp, lang_bp2)


def _pool_body(tf_hbm, tl_hbm, sids_hbm, iids_hbm,
               pf_hbm, insf_hbm, pol_hbm, insl_hbm,
               ids_v, iid_v, rows_v, inst_v, out_v, sem0, sem1):
    wid = lax.axis_index("s") * _NC + lax.axis_index("c")
    base = wid * _BPW
    pltpu.sync_copy(sids_hbm.at[pl.ds(base * _LP, _BPW * _LP)], ids_v)
    pltpu.sync_copy(iids_hbm.at[pl.ds(base, _BPW)], iid_v)
    for c in range(_BPW // _LANES):
        v = iid_v[pl.ds(c * _LANES, _LANES)]
        iid_v[pl.ds(c * _LANES, _LANES)] = jnp.where(
            v == 0, jnp.full((_LANES,), _PAD_ROW, jnp.int32), v)

    sems = (sem0, sem1)

    for t_hbm, p_hbm, i_hbm in ((tf_hbm, pf_hbm, insf_hbm),
                                (tl_hbm, pol_hbm, insl_hbm)):
        pltpu.async_copy(t_hbm.at[iid_v], inst_v, sem0).wait()
        pltpu.sync_copy(inst_v, i_hbm.at[pl.ds(base, _BPW)])

        def _start(g, b):
            off = pl.multiple_of(g * _CIDS, _CIDS)
            pltpu.async_copy(
                t_hbm.at[ids_v.at[pl.ds(off, _CIDS)]], rows_v.at[b], sems[b])

        _start(0, 0)
        _start(1, 1)

        def chunk_body(gg, carry):
            for b in range(2):
                g = gg * 2 + b
                pltpu.make_async_copy(
                    t_hbm.at[ids_v.at[pl.ds(0, _CIDS)]],
                    rows_v.at[b], sems[b]).wait()

                def acc_body(l, accs):
                    new = []
                    for r in range(_CB):
                        for c in range(_HID // _LANES):
                            new.append(accs[r * 8 + c] +
                                       rows_v[b, r * _LP + l,
                                              pl.ds(c * _LANES, _LANES)])
                    return tuple(new)

                accs = lax.fori_loop(
                    0, _LP, acc_body,
                    tuple(jnp.zeros((_LANES,), jnp.float32) for _ in range(16)))
                for r in range(_CB):
                    for c in range(_HID // _LANES):
                        out_v[g * _CB + r, pl.ds(c * _LANES, _LANES)] = \
                            accs[r * 8 + c]
                ng = g + 2

                @pl.when(ng < _CHUNKS)
                def _():
                    _start(ng, b)
            return carry

        lax.fori_loop(0, _CHUNKS // 2, chunk_body, 0)
        pltpu.sync_copy(out_v, p_hbm.at[pl.ds(base, _BPW)])


def _pool(Tf, Tl, sids, iids):
    mesh = plsc.VectorSubcoreMesh(
        core_axis_name="c", subcore_axis_name="s",
        num_cores=_NC, num_subcores=_NS)
    f = pl.kernel(
        _pool_body,
        out_type=[jax.ShapeDtypeStruct((_B, _HID), jnp.float32)] * 4,
        mesh=mesh,
        scratch_types=[
            pltpu.VMEM((_BPW * _LP,), jnp.int32),
            pltpu.VMEM((_BPW,), jnp.int32),
            pltpu.VMEM((2, _CIDS, _HID), jnp.float32),
            pltpu.VMEM((_BPW, _HID), jnp.float32),
            pltpu.VMEM((_BPW, _HID), jnp.float32),
            pltpu.SemaphoreType.DMA,
            pltpu.SemaphoreType.DMA,
        ],
    )
    return f(Tf, Tl, sids, iids)


def _scorer_body(pf, insf, pol, insl, w1f, b1f, w2f, w1l, b1l, w2l, out):
    sf = jnp.maximum(jnp.dot(pf[...], w1f[...],
                             preferred_element_type=jnp.float32) + b1f[...], 0.0)
    s2f = jnp.maximum(jnp.dot(pf[...] + insf[...], w1f[...],
                              preferred_element_type=jnp.float32) + b1f[...], 0.0)
    df = jnp.dot(s2f - sf, w2f[...], preferred_element_type=jnp.float32)
    sl = jnp.maximum(jnp.dot(pol[...], w1l[...],
                             preferred_element_type=jnp.float32) + b1l[...], 0.0)
    s2l = jnp.maximum(jnp.dot(pol[...] + insl[...], w1l[...],
                              preferred_element_type=jnp.float32) + b1l[...], 0.0)
    dl = jnp.dot(s2l - sl, w2l[...], preferred_element_type=jnp.float32)
    out[...] = jnp.concatenate([df, dl], axis=1) * (1.0 / _TEMP)


def _scorer(pf, insf, pol, insl, w1f, b1f2, w2f, w1l, b1l2, w2l):
    blk = 512
    nblk = _B // blk
    emb_spec = pl.BlockSpec((blk, _HID), lambda i: (i, 0))
    return pl.pallas_call(
        _scorer_body,
        grid=(nblk,),
        in_specs=[
            emb_spec, emb_spec, emb_spec, emb_spec,
            pl.BlockSpec((_HID, _COMB), lambda i: (0, 0)),
            pl.BlockSpec((1, _COMB), lambda i: (0, 0)),
            pl.BlockSpec((_COMB, 1), lambda i: (0, 0)),
            pl.BlockSpec((_HID, _COMB), lambda i: (0, 0)),
            pl.BlockSpec((1, _COMB), lambda i: (0, 0)),
            pl.BlockSpec((_COMB, 1), lambda i: (0, 0)),
        ],
        out_specs=pl.BlockSpec((blk, 2), lambda i: (i, 0)),
        out_shape=jax.ShapeDtypeStruct((_B, 2), jnp.float32),
    )(pf, insf, pol, insl, w1f, b1f2, w2f, w1l, b1l2, w2l)


def kernel(set_ids, inst_ids,
           field_table, field_Wp, field_bp, field_Ws1, field_bs1, field_Ws2,
           field_bs2,
           lang_table, lang_Wp, lang_bp, lang_Ws1, lang_bs1, lang_Ws2,
           lang_bs2):
    sids = jnp.pad(set_ids, ((0, 0), (0, _LP - _L))).reshape(-1)
    iids = inst_ids.reshape(-1)
    Tf, Tl = _transform(field_table, field_Wp, field_bp.reshape(1, _HID),
                        lang_table, lang_Wp, lang_bp.reshape(1, _HID))
    pf, insf, pol, insl = _pool(Tf, Tl, sids, iids)
    out2 = _scorer(pf, insf, pol, insl,
                   field_Ws1, field_bs1.reshape(1, _COMB), field_Ws2,
                   lang_Ws1, lang_bs1.reshape(1, _COMB), lang_Ws2)
    return out2.T


# speedup vs baseline: 2.3931x; 2.3931x over previous
import functools

import jax
import jax.numpy as jnp
from jax import lax
from jax.experimental import pallas as pl
from jax.experimental.pallas import tpu as pltpu
from jax.experimental.pallas import tpu_sc as plsc

_VOCAB = 100000
_EMB = 64
_FIELD_D = 256
_HID = 128
_COMB = 64
_B = 4096
_L = 50
_TEMP = 0.5

_ROWBLK = 1024
_NBLK = 98
_NPAD = _NBLK * _ROWBLK
_PAD_ROW = _VOCAB + 1

_NC, _NS, _LANES = 2, 16, 16
_NW = _NC * _NS
_BPW = _B // _NW
_LP = 56
_CB = 2
_CHUNKS = _BPW // _CB
_CIDS = _CB * _LP


def _bdot(a, b):
    return jnp.dot(a.astype(jnp.bfloat16), b.astype(jnp.bfloat16),
                   preferred_element_type=jnp.float32)


def _transform_body(tf_in, wf, bf, tl_in, wl, bl, tf_out, tl_out):
    i = pl.program_id(0)
    row = i * _ROWBLK + lax.broadcasted_iota(jnp.int32, (_ROWBLK, 1), 0)
    valid = row <= _VOCAB
    nonzero = row != 0
    xf = jnp.where(valid, tf_in[...], 0.0)
    hf = _bdot(xf, wf[...]) + bf[...]
    tf_out[...] = jnp.where(nonzero, jnp.maximum(hf, 0.0), 0.0)
    xl = jnp.where(valid, tl_in[...], 0.0)
    hl = _bdot(xl, wl[...]) + bl[...]
    tl_out[...] = jnp.where(nonzero, jnp.maximum(hl, 0.0), 0.0)


def _transform(field_table, field_Wp, field_bp2, lang_table, lang_Wp, lang_bp2):
    return pl.pallas_call(
        _transform_body,
        grid=(_NBLK,),
        in_specs=[
            pl.BlockSpec((_ROWBLK, _FIELD_D), lambda i: (i, 0)),
            pl.BlockSpec((_FIELD_D, _HID), lambda i: (0, 0)),
            pl.BlockSpec((1, _HID), lambda i: (0, 0)),
            pl.BlockSpec((_ROWBLK, _EMB), lambda i: (i, 0)),
            pl.BlockSpec((_EMB, _HID), lambda i: (0, 0)),
            pl.BlockSpec((1, _HID), lambda i: (0, 0)),
        ],
        out_specs=[
            pl.BlockSpec((_ROWBLK, _HID), lambda i: (i, 0)),
            pl.BlockSpec((_ROWBLK, _HID), lambda i: (i, 0)),
        ],
        out_shape=[
            jax.ShapeDtypeStruct((_NPAD, _HID), jnp.float32),
            jax.ShapeDtypeStruct((_NPAD, _HID), jnp.float32),
        ],
    )(field_table, field_Wp, field_bp2, lang_table, lang_Wp, lang_bp2)


def _pool_body(tf_hbm, tl_hbm, sids_hbm, iids_hbm,
               pf_hbm, insf_hbm, pol_hbm, insl_hbm,
               ids_v, iid_v, rows_v, inst_v, out_v, sem0, sem1):
    wid = lax.axis_index("s") * _NC + lax.axis_index("c")
    base = wid * _BPW
    pltpu.sync_copy(sids_hbm.at[pl.ds(base * _LP, _BPW * _LP)], ids_v)
    pltpu.sync_copy(iids_hbm.at[pl.ds(base, _BPW)], iid_v)
    for c in range(_BPW // _LANES):
        v = iid_v[pl.ds(c * _LANES, _LANES)]
        iid_v[pl.ds(c * _LANES, _LANES)] = jnp.where(
            v == 0, jnp.full((_LANES,), _PAD_ROW, jnp.int32), v)

    sems = (sem0, sem1)

    for t_hbm, p_hbm, i_hbm in ((tf_hbm, pf_hbm, insf_hbm),
                                (tl_hbm, pol_hbm, insl_hbm)):
        pltpu.async_copy(t_hbm.at[iid_v], inst_v, sem0).wait()
        pltpu.sync_copy(inst_v, i_hbm.at[pl.ds(base, _BPW)])

        def _start(g, b):
            off = pl.multiple_of(g * _CIDS, _CIDS)
            pltpu.async_copy(
                t_hbm.at[ids_v.at[pl.ds(off, _CIDS)]], rows_v.at[b], sems[b])

        _start(0, 0)
        _start(1, 1)

        def chunk_body(gg, carry):
            for b in range(2):
                g = gg * 2 + b
                pltpu.make_async_copy(
                    t_hbm.at[ids_v.at[pl.ds(0, _CIDS)]],
                    rows_v.at[b], sems[b]).wait()

                def acc_body(l, accs):
                    new = []
                    for r in range(_CB):
                        for c in range(_HID // _LANES):
                            new.append(accs[r * 8 + c] +
                                       rows_v[b, r * _LP + l,
                                              pl.ds(c * _LANES, _LANES)])
                    return tuple(new)

                accs = lax.fori_loop(
                    0, _LP, acc_body,
                    tuple(jnp.zeros((_LANES,), jnp.float32) for _ in range(16)))
                for r in range(_CB):
                    for c in range(_HID // _LANES):
                        out_v[g * _CB + r, pl.ds(c * _LANES, _LANES)] = \
                            accs[r * 8 + c]
                ng = g + 2

                @pl.when(ng < _CHUNKS)
                def _():
                    _start(ng, b)
            return carry

        lax.fori_loop(0, _CHUNKS // 2, chunk_body, 0)
        pltpu.sync_copy(out_v, p_hbm.at[pl.ds(base, _BPW)])


def _pool(Tf, Tl, sids, iids):
    mesh = plsc.VectorSubcoreMesh(
        core_axis_name="c", subcore_axis_name="s",
        num_cores=_NC, num_subcores=_NS)
    f = pl.kernel(
        _pool_body,
        out_type=[jax.ShapeDtypeStruct((_B, _HID), jnp.float32)] * 4,
        mesh=mesh,
        scratch_types=[
            pltpu.VMEM((_BPW * _LP,), jnp.int32),
            pltpu.VMEM((_BPW,), jnp.int32),
            pltpu.VMEM((2, _CIDS, _HID), jnp.float32),
            pltpu.VMEM((_BPW, _HID), jnp.float32),
            pltpu.VMEM((_BPW, _HID), jnp.float32),
            pltpu.SemaphoreType.DMA,
            pltpu.SemaphoreType.DMA,
        ],
    )
    return f(Tf, Tl, sids, iids)


def _scorer_body(pf, insf, pol, insl, w1f, b1f, w2f, w1l, b1l, w2l, out):
    hf = jnp.maximum(_bdot(pf[...], w1f[...]) + b1f[...], 0.0)
    h2f = jnp.maximum(_bdot(pf[...] + insf[...], w1f[...]) + b1f[...], 0.0)
    df = _bdot(h2f, w2f[...]) - _bdot(hf, w2f[...])
    hl = jnp.maximum(_bdot(pol[...], w1l[...]) + b1l[...], 0.0)
    h2l = jnp.maximum(_bdot(pol[...] + insl[...], w1l[...]) + b1l[...], 0.0)
    dl = _bdot(h2l, w2l[...]) - _bdot(hl, w2l[...])
    out[...] = jnp.concatenate([df, dl], axis=1) * (1.0 / _TEMP)


def _scorer(pf, insf, pol, insl, w1f, b1f2, w2f, w1l, b1l2, w2l):
    blk = 512
    nblk = _B // blk
    emb_spec = pl.BlockSpec((blk, _HID), lambda i: (i, 0))
    return pl.pallas_call(
        _scorer_body,
        grid=(nblk,),
        in_specs=[
            emb_spec, emb_spec, emb_spec, emb_spec,
            pl.BlockSpec((_HID, _COMB), lambda i: (0, 0)),
            pl.BlockSpec((1, _COMB), lambda i: (0, 0)),
            pl.BlockSpec((_COMB, 1), lambda i: (0, 0)),
            pl.BlockSpec((_HID, _COMB), lambda i: (0, 0)),
            pl.BlockSpec((1, _COMB), lambda i: (0, 0)),
            pl.BlockSpec((_COMB, 1), lambda i: (0, 0)),
        ],
        out_specs=pl.BlockSpec((blk, 2), lambda i: (i, 0)),
        out_shape=jax.ShapeDtypeStruct((_B, 2), jnp.float32),
    )(pf, insf, pol, insl, w1f, b1f2, w2f, w1l, b1l2, w2l)


def kernel(set_ids, inst_ids,
           field_table, field_Wp, field_bp, field_Ws1, field_bs1, field_Ws2,
           field_bs2,
           lang_table, lang_Wp, lang_bp, lang_Ws1, lang_bs1, lang_Ws2,
           lang_bs2):
    sids = jnp.pad(set_ids, ((0, 0), (0, _LP - _L))).reshape(-1)
    iids = inst_ids.reshape(-1)
    Tf, Tl = _transform(field_table, field_Wp, field_bp.reshape(1, _HID),
                        lang_table, lang_Wp, lang_bp.reshape(1, _HID))
    pf, insf, pol, insl = _pool(Tf, Tl, sids, iids)
    out2 = _scorer(pf, insf, pol, insl,
                   field_Ws1, field_bs1.reshape(1, _COMB), field_Ws2,
                   lang_Ws1, lang_bs1.reshape(1, _COMB), lang_Ws2)
    return out2.T
